# baseline (speedup 1.0000x reference)
import jax
import jax.numpy as jnp
from jax.experimental import pallas as pl
from jax.experimental.pallas import tpu as pltpu
from jax.sharding import Mesh, NamedSharding, PartitionSpec as P


def _packed_ffn_kernel(x_ref, w1_ref, w2_ref, o_ref):
    R = x_ref.shape[0]
    w1 = w1_ref[...].astype(jnp.bfloat16)
    w2 = w2_ref[...].astype(jnp.bfloat16)
    z2 = jnp.zeros_like(w2)
    w2p = jnp.concatenate(
        [jnp.concatenate([w2, z2], axis=1), jnp.concatenate([z2, w2], axis=1)],
        axis=0,
    )
    x = x_ref[...].astype(jnp.bfloat16)
    h = jnp.tanh(jnp.dot(x, w1, preferred_element_type=jnp.float32))
    hp = jnp.concatenate(
        [h[: R // 2].astype(jnp.bfloat16), h[R // 2 :].astype(jnp.bfloat16)],
        axis=1,
    )
    y = jnp.dot(hp, w2p, preferred_element_type=jnp.float32)
    o_ref[: R // 2, :] = y[:, :128]
    o_ref[R // 2 :, :] = y[:, 128:]


def _ffn_pallas_call(label_emb, w1_cat, w2_bd):
    L, E = label_emb.shape
    HE = w1_cat.shape[1]

    R = 16384
    while L % R:
        R //= 2

    return pl.pallas_call(
        _packed_ffn_kernel if R % 2 == 0 else _unpacked_ffn_kernel,
        out_shape=jax.ShapeDtypeStruct((L, HE), label_emb.dtype),
        grid=(L // R,),
        in_specs=[
            pl.BlockSpec((R, E), lambda i: (i, 0)),
            pl.BlockSpec((E, HE), lambda i: (0, 0)),
            pl.BlockSpec((HE, HE), lambda i: (0, 0)),
        ],
        out_specs=pl.BlockSpec((R, HE), lambda i: (i, 0)),
        compiler_params=pltpu.CompilerParams(dimension_semantics=("parallel",)),
        cost_estimate=pl.CostEstimate(
            flops=2 * L * E * HE + 2 * L * HE * HE,
            transcendentals=L * HE,
            bytes_accessed=(L * E + L * HE) * 4 + (E * HE + HE * HE) * 4,
        ),
    )(label_emb, w1_cat, w2_bd)


def kernel(label_emb, w1_cat, w2_bd):
    L = label_emb.shape[0]
    devs = jax.devices()
    n = 2 if (len(devs) >= 2 and L % 2 == 0) else 1
    if n == 1:
        return _ffn_pallas_call(label_emb, w1_cat, w2_bd)

    mesh = Mesh(devs[:n], ("x",))
    xs = jax.device_put(label_emb, NamedSharding(mesh, P("x", None)))
    w1s = jax.device_put(w1_cat, NamedSharding(mesh, P(None, None)))
    w2s = jax.device_put(w2_bd, NamedSharding(mesh, P(None, None)))
    f = jax.shard_map(
        _ffn_pallas_call,
        mesh=mesh,
        in_specs=(P("x", None), P(None, None), P(None, None)),
        out_specs=P("x", None),
        check_vma=False,
    )
    return f(xs, w1s, w2s)


def _unpacked_ffn_kernel(x_ref, w1_ref, w2_ref, o_ref):
    w1 = w1_ref[...].astype(jnp.bfloat16)
    w2 = w2_ref[...].astype(jnp.bfloat16)
    x = x_ref[...].astype(jnp.bfloat16)
    h = jnp.tanh(jnp.dot(x, w1, preferred_element_type=jnp.float32))
    o_ref[...] = jnp.dot(h.astype(jnp.bfloat16), w2, preferred_element_type=jnp.float32)


# speedup vs baseline: 1.8237x; 1.8237x over previous
import jax
import jax.numpy as jnp
from jax.experimental import pallas as pl
from jax.experimental.pallas import tpu as pltpu


_R = 16384


def _packed_ffn_kernel(x_ref, w1_ref, w2_ref, o_ref):
    R = o_ref.shape[0]
    w1 = w1_ref[...].astype(jnp.bfloat16)
    w2 = w2_ref[...].astype(jnp.bfloat16)
    z1 = jnp.zeros_like(w1)
    z2 = jnp.zeros_like(w2)
    w1p = jnp.concatenate(
        [jnp.concatenate([w1, z1], axis=1), jnp.concatenate([z1, w1], axis=1)],
        axis=0,
    )
    w2p = jnp.concatenate(
        [jnp.concatenate([w2, z2], axis=1), jnp.concatenate([z2, w2], axis=1)],
        axis=0,
    )
    xp = x_ref[...].astype(jnp.bfloat16)
    h = jnp.tanh(jnp.dot(xp, w1p, preferred_element_type=jnp.float32))
    y = jnp.dot(h.astype(jnp.bfloat16), w2p, preferred_element_type=jnp.float32)
    o_ref[: R // 2, :] = y[:, :128]
    o_ref[R // 2 :, :] = y[:, 128:]


def _fold_pairs(label_emb, R):
    L, E = label_emb.shape
    x3 = label_emb.reshape(L // R, R, E)
    xf = jnp.concatenate([x3[:, : R // 2], x3[:, R // 2 :]], axis=2)
    return xf.reshape(L // 2, 2 * E)


def kernel(label_emb, w1_cat, w2_bd):
    L, E = label_emb.shape
    HE = w1_cat.shape[1]

    R = _R
    while L % R:
        R //= 2
    if R < 2:
        return _unpacked_call(label_emb, w1_cat, w2_bd)

    xp = _fold_pairs(label_emb, R)
    out = pl.pallas_call(
        _packed_ffn_kernel,
        out_shape=jax.ShapeDtypeStruct((L, HE), label_emb.dtype),
        grid=(L // R,),
        in_specs=[
            pl.BlockSpec((R // 2, 2 * E), lambda i: (i, 0)),
            pl.BlockSpec((E, HE), lambda i: (0, 0)),
            pl.BlockSpec((HE, HE), lambda i: (0, 0)),
        ],
        out_specs=pl.BlockSpec((R, HE), lambda i: (i, 0)),
        compiler_params=pltpu.CompilerParams(dimension_semantics=("parallel",)),
        cost_estimate=pl.CostEstimate(
            flops=2 * L * E * HE + 2 * L * HE * HE,
            transcendentals=L * HE,
            bytes_accessed=(L * E + L * HE) * 4 + (E * HE + HE * HE) * 4,
        ),
    )(xp, w1_cat, w2_bd)
    return out


def _unpacked_ffn_kernel(x_ref, w1_ref, w2_ref, o_ref):
    w1 = w1_ref[...].astype(jnp.bfloat16)
    w2 = w2_ref[...].astype(jnp.bfloat16)
    x = x_ref[...].astype(jnp.bfloat16)
    h = jnp.tanh(jnp.dot(x, w1, preferred_element_type=jnp.float32))
    o_ref[...] = jnp.dot(h.astype(jnp.bfloat16), w2, preferred_element_type=jnp.float32)


def _unpacked_call(label_emb, w1_cat, w2_bd):
    L, E = label_emb.shape
    HE = w1_cat.shape[1]
    return pl.pallas_call(
        _unpacked_ffn_kernel,
        out_shape=jax.ShapeDtypeStruct((L, HE), label_emb.dtype),
        in_specs=[pl.BlockSpec(memory_space=pltpu.MemorySpace.VMEM)] * 3,
        out_specs=pl.BlockSpec(memory_space=pltpu.MemorySpace.VMEM),
    )(label_emb, w1_cat, w2_bd)


# speedup vs baseline: 2.8595x; 1.5680x over previous
import jax
import jax.numpy as jnp
from jax.experimental import pallas as pl
from jax.experimental.pallas import tpu as pltpu
from jax.sharding import Mesh, NamedSharding, PartitionSpec as P


def _packed_ffn_kernel(xa_ref, xb_ref, xc_ref, xd_ref, w1_ref, w2_ref, o_ref):
    R = 4 * xa_ref.shape[0]
    w1 = w1_ref[...].astype(jnp.bfloat16)
    w2 = w2_ref[...].astype(jnp.bfloat16)
    z2 = jnp.zeros_like(w2)
    w2p = jnp.concatenate(
        [jnp.concatenate([w2, z2], axis=1), jnp.concatenate([z2, w2], axis=1)],
        axis=0,
    )
    x = jnp.concatenate(
        [xa_ref[...], xb_ref[...], xc_ref[...], xd_ref[...]], axis=0
    ).astype(jnp.bfloat16)
    h = jnp.tanh(jnp.dot(x, w1, preferred_element_type=jnp.float32))
    hp = jnp.concatenate(
        [h[: R // 2].astype(jnp.bfloat16), h[R // 2 :].astype(jnp.bfloat16)],
        axis=1,
    )
    y = jnp.dot(hp, w2p, preferred_element_type=jnp.float32)
    o_ref[: R // 2, :] = y[:, :128]
    o_ref[R // 2 :, :] = y[:, 128:]


def _ffn_pallas_call(label_emb, w1_cat, w2_bd):
    L, E = label_emb.shape
    HE = w1_cat.shape[1]

    R = 16384
    while L % R:
        R //= 2

    return pl.pallas_call(
        _packed_ffn_kernel,
        out_shape=jax.ShapeDtypeStruct((L, HE), label_emb.dtype),
        grid=(L // R,),
        in_specs=[
            pl.BlockSpec((R // 4, E), lambda i: (4 * i, 0)),
            pl.BlockSpec((R // 4, E), lambda i: (4 * i + 1, 0)),
            pl.BlockSpec((R // 4, E), lambda i: (4 * i + 2, 0)),
            pl.BlockSpec((R // 4, E), lambda i: (4 * i + 3, 0)),
            pl.BlockSpec((E, HE), lambda i: (0, 0)),
            pl.BlockSpec((HE, HE), lambda i: (0, 0)),
        ],
        out_specs=pl.BlockSpec((R, HE), lambda i: (i, 0)),
        compiler_params=pltpu.CompilerParams(dimension_semantics=("parallel",)),
        cost_estimate=pl.CostEstimate(
            flops=2 * L * E * HE + 2 * L * HE * HE,
            transcendentals=L * HE,
            bytes_accessed=(L * E + L * HE) * 4 + (E * HE + HE * HE) * 4,
        ),
    )(label_emb, label_emb, label_emb, label_emb, w1_cat, w2_bd)


def kernel(label_emb, w1_cat, w2_bd):
    return _ffn_pallas_call(label_emb, w1_cat, w2_bd)


def _unpacked_ffn_kernel(x_ref, w1_ref, w2_ref, o_ref):
    w1 = w1_ref[...].astype(jnp.bfloat16)
    w2 = w2_ref[...].astype(jnp.bfloat16)
    x = x_ref[...].astype(jnp.bfloat16)
    h = jnp.tanh(jnp.dot(x, w1, preferred_element_type=jnp.float32))
    o_ref[...] = jnp.dot(h.astype(jnp.bfloat16), w2, preferred_element_type=jnp.float32)


# speedup vs baseline: 2.8723x; 1.0045x over previous
import jax
import jax.numpy as jnp
from jax.experimental import pallas as pl
from jax.experimental.pallas import tpu as pltpu


def _packed_ffn_kernel(x_ref, w1_ref, w2_ref, hbm_o_ref, vb_ref, sem_ref):
    i = pl.program_id(0)
    n = pl.num_programs(0)
    R = x_ref.shape[0]
    slot = jax.lax.rem(i, 2)

    def _copy(step, s):
        return pltpu.make_async_copy(
            vb_ref.at[s],
            hbm_o_ref.at[pl.ds(step * R, R), :],
            sem_ref.at[s],
        )

    @pl.when(i >= 2)
    def _():
        _copy(i - 2, slot).wait()

    w1 = w1_ref[...].astype(jnp.bfloat16)
    w2 = w2_ref[...].astype(jnp.bfloat16)
    z2 = jnp.zeros_like(w2)
    w2p = jnp.concatenate(
        [jnp.concatenate([w2, z2], axis=1), jnp.concatenate([z2, w2], axis=1)],
        axis=0,
    )
    x = x_ref[...].astype(jnp.bfloat16)
    h = jnp.tanh(jnp.dot(x, w1, preferred_element_type=jnp.float32))
    hp = jnp.concatenate(
        [h[: R // 2].astype(jnp.bfloat16), h[R // 2 :].astype(jnp.bfloat16)],
        axis=1,
    )
    y = jnp.dot(hp, w2p, preferred_element_type=jnp.float32)
    vb_ref[slot, : R // 2, :] = y[:, :128]
    vb_ref[slot, R // 2 :, :] = y[:, 128:]
    _copy(i, slot).start()

    @pl.when(i == n - 1)
    def _():
        @pl.when(n >= 2)
        def _():
            _copy(n - 2, jax.lax.rem(n - 2, 2)).wait()

        _copy(n - 1, jax.lax.rem(n - 1, 2)).wait()


def kernel(label_emb, w1_cat, w2_bd):
    L, E = label_emb.shape
    HE = w1_cat.shape[1]

    R = 16384
    while L % R:
        R //= 2

    return pl.pallas_call(
        _packed_ffn_kernel,
        out_shape=jax.ShapeDtypeStruct((L, HE), label_emb.dtype),
        grid=(L // R,),
        in_specs=[
            pl.BlockSpec((R, E), lambda i: (i, 0)),
            pl.BlockSpec((E, HE), lambda i: (0, 0)),
            pl.BlockSpec((HE, HE), lambda i: (0, 0)),
        ],
        out_specs=pl.BlockSpec(memory_space=pltpu.MemorySpace.HBM),
        scratch_shapes=[
            pltpu.VMEM((2, R, HE), jnp.float32),
            pltpu.SemaphoreType.DMA((2,)),
        ],
        compiler_params=pltpu.CompilerParams(dimension_semantics=("arbitrary",)),
        cost_estimate=pl.CostEstimate(
            flops=2 * L * E * HE + 2 * L * HE * HE,
            transcendentals=L * HE,
            bytes_accessed=(L * E + L * HE) * 4 + (E * HE + HE * HE) * 4,
        ),
    )(label_emb, w1_cat, w2_bd)
